# baseline (speedup 1.0000x reference)
import functools

import jax
import jax.numpy as jnp
from jax import lax
from jax.experimental import pallas as pl
from jax.experimental.pallas import tpu as pltpu
from jax.experimental.pallas import tpu_sc as plsc

B, L, D_BERT = 1024, 50, 768
ED = 32
N_TOKENS = B * L


CHUNK = 80
NCHUNK = 20


def _sc_gather_body(t0_hbm, t1_hbm, f0_hbm, f1_hbm, g0_hbm, g1_hbm,
                    idx_v, rows_v, sem, *, b_per_w, num_cores):
    wid = lax.axis_index("s") * num_cores + lax.axis_index("c")
    base = wid * b_per_w
    for t_hbm, f_hbm, g_hbm in ((t0_hbm, f0_hbm, g0_hbm),
                                (t1_hbm, f1_hbm, g1_hbm)):
        pltpu.sync_copy(f_hbm.at[wid], idx_v)
        copies = [pltpu.async_copy(t_hbm.at[idx_v.at[j]], rows_v.at[j], sem)
                  for j in range(NCHUNK)]
        for c in copies:
            c.wait()
        for j in range(NCHUNK):
            pltpu.sync_copy(
                rows_v.at[j], g_hbm.at[pl.ds(base + j * CHUNK, CHUNK)])


def _sc_gather(table0, table1, f0, f1):
    info = plsc.get_sparse_core_info()
    nw = info.num_cores * info.num_subcores
    b_per_w = N_TOKENS // nw
    assert b_per_w == NCHUNK * CHUNK
    mesh = plsc.VectorSubcoreMesh(core_axis_name="c", subcore_axis_name="s")
    body = functools.partial(_sc_gather_body, b_per_w=b_per_w,
                             num_cores=info.num_cores)
    return pl.kernel(
        body,
        out_type=(
            jax.ShapeDtypeStruct((N_TOKENS, ED), jnp.float32),
            jax.ShapeDtypeStruct((N_TOKENS, ED), jnp.float32),
        ),
        mesh=mesh,
        scratch_types=[
            pltpu.VMEM((NCHUNK, CHUNK), jnp.int32),
            pltpu.VMEM((NCHUNK, CHUNK, ED), jnp.float32),
            pltpu.SemaphoreType.DMA,
        ],
        compiler_params=pltpu.CompilerParams(use_tc_tiling_on_sc=False),
    )(table0, table1, f0, f1)


def _concat_body(bert_ref, g0_ref, g1_ref, out_ref):
    out_ref[...] = jnp.concatenate(
        [bert_ref[...], g0_ref[...], g1_ref[...]], axis=1)


def _tc_concat(bert2d, g0, g1):
    br = 512
    grid = (N_TOKENS // br,)
    return pl.pallas_call(
        _concat_body,
        grid=grid,
        in_specs=[
            pl.BlockSpec((br, D_BERT), lambda i: (i, 0)),
            pl.BlockSpec((br, ED), lambda i: (i, 0)),
            pl.BlockSpec((br, ED), lambda i: (i, 0)),
        ],
        out_specs=pl.BlockSpec((br, D_BERT + 2 * ED), lambda i: (i, 0)),
        out_shape=jax.ShapeDtypeStruct((N_TOKENS, D_BERT + 2 * ED),
                                       jnp.float32),
    )(bert2d, g0, g1)


@jax.jit
def kernel(bert_embed, features0, features1, table0, table1):
    nw = N_TOKENS // (NCHUNK * CHUNK)
    f0 = features0.reshape(nw, NCHUNK, CHUNK).astype(jnp.int32)
    f1 = features1.reshape(nw, NCHUNK, CHUNK).astype(jnp.int32)
    g0, g1 = _sc_gather(table0, table1, f0, f1)
    bert2d = bert_embed.reshape(N_TOKENS, D_BERT)
    out = _tc_concat(bert2d, g0, g1)
    return out.reshape(B, L, D_BERT + 2 * ED)
